# baseline (speedup 1.0000x reference)
import functools

import jax
import jax.numpy as jnp
from jax import lax
from jax.experimental import pallas as pl
from jax.experimental.pallas import tpu as pltpu
from jax.experimental.pallas import tpu_sc as plsc

N_EXPERTS = 4
BLK = 256
NW = 32
G = 128


def _sc_scatter_sort(aev_flat, dest, molv):
    N, D = aev_flat.shape
    per_w = N // NW
    n_g = per_w // G
    dest3 = dest.reshape(NW, n_g, G)
    molv3 = molv.reshape(NW, n_g, G)
    mesh = plsc.VectorSubcoreMesh(core_axis_name="c", subcore_axis_name="s")

    @functools.partial(
        pl.kernel,
        out_type=[
            jax.ShapeDtypeStruct((N, D), jnp.float32),
            jax.ShapeDtypeStruct((N,), jnp.float32),
        ],
        mesh=mesh,
        scratch_types=[
            pltpu.VMEM((n_g, G), jnp.int32),
            pltpu.VMEM((n_g, G), jnp.float32),
            pltpu.VMEM((G, D), jnp.float32),
            pltpu.SemaphoreType.DMA,
            pltpu.SemaphoreType.DMA,
        ],
    )
    def scatter_kernel(aev_hbm, dest_hbm, mol_hbm, out_hbm, outm_hbm,
                       idx_v, mol_v, buf_v, sem1, sem2):
        wid = lax.axis_index("s") * 2 + lax.axis_index("c")
        base = wid * per_w
        pltpu.sync_copy(dest_hbm.at[wid], idx_v)
        pltpu.sync_copy(mol_hbm.at[wid], mol_v)

        def body(g, carry):
            pltpu.sync_copy(aev_hbm.at[pl.ds(base + g * G, G)], buf_v)
            cp1 = pltpu.async_copy(buf_v, out_hbm.at[idx_v.at[g]], sem1)
            cp2 = pltpu.async_copy(mol_v.at[g], outm_hbm.at[idx_v.at[g]], sem2)
            cp1.wait()
            cp2.wait()
            return carry

        lax.fori_loop(0, n_g, body, 0)

    return scatter_kernel(aev_flat, dest3, molv3)


def _celu(x):
    return jnp.where(x > 0, x, 0.1 * (jnp.exp(jnp.minimum(x, 0.0) * 10.0) - 1.0))


def _tc_moe(sorted_aev, sorted_mol, offsets8, wts, bss, n_mol):
    N, D = sorted_aev.shape
    nb = N // BLK
    mol3 = sorted_mol.reshape(nb, 1, BLK)

    def body(off_ref, x_ref, mol_ref, w1, b1, w2, b2, w3, b3, w4, b4,
             out_ref, yacc):
        i = pl.program_id(0)
        j0 = i * BLK

        @pl.when(i == 0)
        def _init():
            out_ref[...] = jnp.zeros_like(out_ref)

        yacc[...] = jnp.zeros_like(yacc)
        x = x_ref[...].astype(jnp.bfloat16)
        rows = j0 + lax.broadcasted_iota(jnp.int32, (BLK, 1), 0)

        for e in range(N_EXPERTS):
            lo = off_ref[e]
            hi = off_ref[e + 1]

            @pl.when((j0 < hi) & (j0 + BLK > lo))
            def _compute(e=e, lo=lo, hi=hi):
                h = lax.dot_general(x, w1[e], (((1,), (0,)), ((), ())),
                                    preferred_element_type=jnp.float32)
                h = _celu(h + b1[e]).astype(jnp.bfloat16)
                h = lax.dot_general(h, w2[e], (((1,), (0,)), ((), ())),
                                    preferred_element_type=jnp.float32)
                h = _celu(h + b2[e]).astype(jnp.bfloat16)
                h = lax.dot_general(h, w3[e], (((1,), (0,)), ((), ())),
                                    preferred_element_type=jnp.float32)
                h = _celu(h + b3[e]).astype(jnp.bfloat16)
                y = lax.dot_general(h, w4[e], (((1,), (0,)), ((), ())),
                                    preferred_element_type=jnp.float32)
                y = y + b4[e]
                mask = (rows >= lo) & (rows < hi)
                yacc[...] += jnp.where(mask, y, 0.0)

        mol = mol_ref[0].astype(jnp.int32)
        mids = lax.broadcasted_iota(jnp.int32, (n_mol, BLK), 0)
        oh_t = (mids == mol).astype(jnp.bfloat16)
        yb = yacc[...].astype(jnp.bfloat16)
        out_ref[...] += lax.dot_general(
            oh_t, yb, (((1,), (0,)), ((), ())),
            preferred_element_type=jnp.float32)

    out = pl.pallas_call(
        body,
        grid=(nb,),
        in_specs=[
            pl.BlockSpec(memory_space=pltpu.SMEM),
            pl.BlockSpec((BLK, D), lambda i: (i, 0)),
            pl.BlockSpec((1, 1, BLK), lambda i: (i, 0, 0)),
            pl.BlockSpec(wts[0].shape, lambda i: (0, 0, 0)),
            pl.BlockSpec(bss[0].shape, lambda i: (0, 0, 0)),
            pl.BlockSpec(wts[1].shape, lambda i: (0, 0, 0)),
            pl.BlockSpec(bss[1].shape, lambda i: (0, 0, 0)),
            pl.BlockSpec(wts[2].shape, lambda i: (0, 0, 0)),
            pl.BlockSpec(bss[2].shape, lambda i: (0, 0, 0)),
            pl.BlockSpec(wts[3].shape, lambda i: (0, 0, 0)),
            pl.BlockSpec(bss[3].shape, lambda i: (0, 0, 0)),
        ],
        out_specs=pl.BlockSpec((n_mol, 1), lambda i: (0, 0)),
        out_shape=jax.ShapeDtypeStruct((n_mol, 1), jnp.float32),
        scratch_shapes=[pltpu.VMEM((BLK, 1), jnp.float32)],
        compiler_params=pltpu.CompilerParams(
            dimension_semantics=("arbitrary",)),
    )(offsets8, sorted_aev, mol3,
      wts[0], bss[0], wts[1], bss[1], wts[2], bss[2], wts[3], bss[3])
    return out.reshape(n_mol)


def kernel(species, aev, params):
    n_mol, A = species.shape
    D = aev.shape[-1]
    N = n_mol * A

    sf = species.reshape(-1).astype(jnp.int32)
    aev_flat = aev.reshape(N, D)

    one_hot = (sf[:, None] == jnp.arange(N_EXPERTS, dtype=jnp.int32)[None, :]
               ).astype(jnp.int32)
    csum = jnp.cumsum(one_hot, axis=0)
    counts = csum[-1]
    offsets = jnp.concatenate(
        [jnp.zeros((1,), jnp.int32), jnp.cumsum(counts)]).astype(jnp.int32)
    rank = jnp.sum(one_hot * csum, axis=1) - 1
    dest = offsets[sf] + rank
    molv = (jnp.arange(N, dtype=jnp.int32) // A).astype(jnp.float32)

    sorted_aev, sorted_mol = _sc_scatter_sort(aev_flat, dest, molv)

    wts = [jnp.stack([p[l][0].T for p in params]).astype(jnp.bfloat16)
           for l in range(4)]
    bss = [jnp.stack([p[l][1] for p in params])[:, None, :] for l in range(4)]
    offsets8 = jnp.concatenate([offsets, jnp.zeros((3,), jnp.int32)])

    sums = _tc_moe(sorted_aev, sorted_mol, offsets8, wts, bss, n_mol)
    return (species, sums)


# speedup vs baseline: 1.5894x; 1.5894x over previous
import functools

import jax
import jax.numpy as jnp
from jax import lax
from jax.experimental import pallas as pl
from jax.experimental.pallas import tpu as pltpu
from jax.experimental.pallas import tpu_sc as plsc

N_EXPERTS = 4
BLK = 256
NW = 32
G = 128


def _sc_scatter_sort(aev_flat, dest3):
    N, D = aev_flat.shape
    per_w = N // NW
    n_g = per_w // G
    mesh = plsc.VectorSubcoreMesh(core_axis_name="c", subcore_axis_name="s")

    @functools.partial(
        pl.kernel,
        out_type=jax.ShapeDtypeStruct((N, D), jnp.float32),
        mesh=mesh,
        scratch_types=[
            pltpu.VMEM((n_g, G), jnp.int32),
            pltpu.VMEM((G, D), jnp.float32),
            pltpu.VMEM((G, D), jnp.float32),
            pltpu.SemaphoreType.DMA,
            pltpu.SemaphoreType.DMA,
        ],
    )
    def scatter_kernel(aev_hbm, dest_hbm, out_hbm, idx_v, buf0, buf1,
                       sem_in, sem_out):
        wid = lax.axis_index("s") * 2 + lax.axis_index("c")
        base = wid * per_w
        pltpu.sync_copy(dest_hbm.at[wid], idx_v)
        bufs = (buf0, buf1)
        ins = [pltpu.async_copy(aev_hbm.at[pl.ds(base, G)], buf0, sem_in),
               pltpu.async_copy(aev_hbm.at[pl.ds(base + G, G)], buf1, sem_in)]
        outs = []
        for g in range(n_g):
            ins[g].wait()
            outs.append(pltpu.async_copy(
                bufs[g % 2], out_hbm.at[idx_v.at[g]], sem_out))
            if g + 2 < n_g:
                outs[g].wait()
                ins.append(pltpu.async_copy(
                    aev_hbm.at[pl.ds(base + (g + 2) * G, G)],
                    bufs[g % 2], sem_in))
        outs[n_g - 2].wait()
        outs[n_g - 1].wait()

    return scatter_kernel(aev_flat, dest3)


def _sc_segment_sum(y_flat, dest3t, n_mol, atoms_per_mol):
    N = y_flat.shape[0]
    per_w = N // NW
    n_g = per_w // G
    mol_per_w = per_w // atoms_per_mol
    n_grp = mol_per_w // 16
    mesh = plsc.VectorSubcoreMesh(core_axis_name="c", subcore_axis_name="s")

    @functools.partial(
        pl.kernel,
        out_type=jax.ShapeDtypeStruct((n_mol,), jnp.float32),
        mesh=mesh,
        scratch_types=[
            pltpu.VMEM((n_g, G), jnp.int32),
            pltpu.VMEM((n_g, G), jnp.float32),
            pltpu.VMEM((mol_per_w,), jnp.float32),
            pltpu.SemaphoreType.DMA,
        ],
    )
    def segsum_kernel(y_hbm, dest_hbm, out_hbm, idx_v, ybuf, out_v, sem):
        wid = lax.axis_index("s") * 2 + lax.axis_index("c")
        pltpu.sync_copy(dest_hbm.at[wid], idx_v)
        cps = [pltpu.async_copy(y_hbm.at[idx_v.at[g]], ybuf.at[g], sem)
               for g in range(n_g)]
        for cp in cps:
            cp.wait()
        for m in range(n_grp):
            acc = jnp.zeros((16,), jnp.float32)
            for t in range(atoms_per_mol):
                off = m * 16 * atoms_per_mol + t * 16
                acc = acc + ybuf[off // G, pl.ds(off % G, 16)]
            out_v[pl.ds(m * 16, 16)] = acc
        pltpu.sync_copy(out_v, out_hbm.at[pl.ds(wid * mol_per_w, mol_per_w)])

    return segsum_kernel(y_flat, dest3t)


def _celu(x):
    return jnp.where(x > 0, x, 0.1 * (jnp.exp(jnp.minimum(x, 0.0) * 10.0) - 1.0))


def _tc_moe(sorted_aev, offsets8, wts, bss):
    N, D = sorted_aev.shape
    nb = N // BLK

    def body(off_ref, x_ref, w1, b1, w2, b2, w3, b3, w4, b4, out_ref):
        i = pl.program_id(0)
        j0 = i * BLK
        out_ref[...] = jnp.zeros_like(out_ref)
        x = x_ref[...].astype(jnp.bfloat16)
        rows = j0 + lax.broadcasted_iota(jnp.int32, (BLK, 1), 0)

        for e in range(N_EXPERTS):
            lo = off_ref[e]
            hi = off_ref[e + 1]

            @pl.when((j0 < hi) & (j0 + BLK > lo))
            def _compute(e=e, lo=lo, hi=hi):
                h = lax.dot_general(x, w1[e], (((1,), (0,)), ((), ())),
                                    preferred_element_type=jnp.float32)
                h = _celu(h + b1[e]).astype(jnp.bfloat16)
                h = lax.dot_general(h, w2[e], (((1,), (0,)), ((), ())),
                                    preferred_element_type=jnp.float32)
                h = _celu(h + b2[e]).astype(jnp.bfloat16)
                h = lax.dot_general(h, w3[e], (((1,), (0,)), ((), ())),
                                    preferred_element_type=jnp.float32)
                h = _celu(h + b3[e]).astype(jnp.bfloat16)
                y = lax.dot_general(h, w4[e], (((1,), (0,)), ((), ())),
                                    preferred_element_type=jnp.float32)
                y = y + b4[e]
                mask = (rows >= lo) & (rows < hi)
                out_ref[...] += jnp.where(mask, y, 0.0)

    out = pl.pallas_call(
        body,
        grid=(nb,),
        in_specs=[
            pl.BlockSpec(memory_space=pltpu.SMEM),
            pl.BlockSpec((BLK, D), lambda i: (i, 0)),
            pl.BlockSpec(wts[0].shape, lambda i: (0, 0, 0)),
            pl.BlockSpec(bss[0].shape, lambda i: (0, 0, 0)),
            pl.BlockSpec(wts[1].shape, lambda i: (0, 0, 0)),
            pl.BlockSpec(bss[1].shape, lambda i: (0, 0, 0)),
            pl.BlockSpec(wts[2].shape, lambda i: (0, 0, 0)),
            pl.BlockSpec(bss[2].shape, lambda i: (0, 0, 0)),
            pl.BlockSpec(wts[3].shape, lambda i: (0, 0, 0)),
            pl.BlockSpec(bss[3].shape, lambda i: (0, 0, 0)),
        ],
        out_specs=pl.BlockSpec((BLK, 1), lambda i: (i, 0)),
        out_shape=jax.ShapeDtypeStruct((N, 1), jnp.float32),
        compiler_params=pltpu.CompilerParams(
            dimension_semantics=("arbitrary",)),
    )(offsets8, sorted_aev,
      wts[0], bss[0], wts[1], bss[1], wts[2], bss[2], wts[3], bss[3])
    return out


def kernel(species, aev, params):
    n_mol, A = species.shape
    D = aev.shape[-1]
    N = n_mol * A

    sf = species.reshape(-1).astype(jnp.int32)
    aev_flat = aev.reshape(N, D)

    one_hot = (sf[:, None] == jnp.arange(N_EXPERTS, dtype=jnp.int32)[None, :]
               ).astype(jnp.int32)
    csum = jnp.cumsum(one_hot, axis=0)
    counts = csum[-1]
    offsets = jnp.concatenate(
        [jnp.zeros((1,), jnp.int32), jnp.cumsum(counts)]).astype(jnp.int32)
    rank = jnp.sum(one_hot * csum, axis=1) - 1
    dest = offsets[sf] + rank
    dest3 = dest.reshape(NW, (N // NW) // G, G)

    sorted_aev = _sc_scatter_sort(aev_flat, dest3)

    wts = [jnp.stack([p[l][0].T for p in params]).astype(jnp.bfloat16)
           for l in range(4)]
    bss = [jnp.stack([p[l][1] for p in params])[:, None, :] for l in range(4)]
    offsets8 = jnp.concatenate([offsets, jnp.zeros((3,), jnp.int32)])

    y = _tc_moe(sorted_aev, offsets8, wts, bss)
    mol_grp = (N // NW) // (16 * A)
    dest3t = (dest.reshape(NW, mol_grp, 16, A)
              .transpose(0, 1, 3, 2)
              .reshape(NW, (N // NW) // G, G))
    sums = _sc_segment_sum(y.reshape(N), dest3t, n_mol, A)
    return (species, sums)


# speedup vs baseline: 2.1492x; 1.3522x over previous
import functools

import jax
import jax.numpy as jnp
from jax import lax
from jax.experimental import pallas as pl
from jax.experimental.pallas import tpu as pltpu
from jax.experimental.pallas import tpu_sc as plsc

N_EXPERTS = 4
BLK = 512
NW = 32
G = 128


def _sc_scatter_sort(aev_flat, dest3):
    N, D = aev_flat.shape
    per_w = N // NW
    n_g = per_w // G
    mesh = plsc.VectorSubcoreMesh(core_axis_name="c", subcore_axis_name="s")

    @functools.partial(
        pl.kernel,
        out_type=jax.ShapeDtypeStruct((N, D), jnp.float32),
        mesh=mesh,
        scratch_types=[
            pltpu.VMEM((n_g, G), jnp.int32),
            pltpu.VMEM((G, D), jnp.float32),
            pltpu.VMEM((G, D), jnp.float32),
            pltpu.SemaphoreType.DMA,
            pltpu.SemaphoreType.DMA,
        ],
    )
    def scatter_kernel(aev_hbm, dest_hbm, out_hbm, idx_v, buf0, buf1,
                       sem_in, sem_out):
        wid = lax.axis_index("s") * 2 + lax.axis_index("c")
        base = wid * per_w
        pltpu.sync_copy(dest_hbm.at[wid], idx_v)
        bufs = (buf0, buf1)
        ins = [pltpu.async_copy(aev_hbm.at[pl.ds(base, G)], buf0, sem_in),
               pltpu.async_copy(aev_hbm.at[pl.ds(base + G, G)], buf1, sem_in)]
        outs = []
        for g in range(n_g):
            ins[g].wait()
            outs.append(pltpu.async_copy(
                bufs[g % 2], out_hbm.at[idx_v.at[g]], sem_out))
            if g + 2 < n_g:
                outs[g].wait()
                ins.append(pltpu.async_copy(
                    aev_hbm.at[pl.ds(base + (g + 2) * G, G)],
                    bufs[g % 2], sem_in))
        outs[n_g - 2].wait()
        outs[n_g - 1].wait()

    return scatter_kernel(aev_flat, dest3)


def _sc_segment_sum(y_flat, dest3t, n_mol, atoms_per_mol):
    N = y_flat.shape[0]
    per_w = N // NW
    n_g = per_w // G
    mol_per_w = per_w // atoms_per_mol
    n_grp = mol_per_w // 16
    mesh = plsc.VectorSubcoreMesh(core_axis_name="c", subcore_axis_name="s")

    @functools.partial(
        pl.kernel,
        out_type=jax.ShapeDtypeStruct((n_mol,), jnp.float32),
        mesh=mesh,
        scratch_types=[
            pltpu.VMEM((n_g, G), jnp.int32),
            pltpu.VMEM((n_g, G), jnp.float32),
            pltpu.VMEM((mol_per_w,), jnp.float32),
            pltpu.SemaphoreType.DMA,
        ],
    )
    def segsum_kernel(y_hbm, dest_hbm, out_hbm, idx_v, ybuf, out_v, sem):
        wid = lax.axis_index("s") * 2 + lax.axis_index("c")
        pltpu.sync_copy(dest_hbm.at[wid], idx_v)
        cps = [pltpu.async_copy(y_hbm.at[idx_v.at[g]], ybuf.at[g], sem)
               for g in range(n_g)]
        for cp in cps:
            cp.wait()
        for m in range(n_grp):
            acc = jnp.zeros((16,), jnp.float32)
            for t in range(atoms_per_mol):
                off = m * 16 * atoms_per_mol + t * 16
                acc = acc + ybuf[off // G, pl.ds(off % G, 16)]
            out_v[pl.ds(m * 16, 16)] = acc
        pltpu.sync_copy(out_v, out_hbm.at[pl.ds(wid * mol_per_w, mol_per_w)])

    return segsum_kernel(y_flat, dest3t)


def _celu(x):
    p = jnp.exp2(x * 14.426950408889634)
    return jnp.where(x > 0, x, p * 0.1 - 0.1)


def _tc_moe(sorted_aev, offsets8, wts, bss):
    N, D = sorted_aev.shape
    nb = N // BLK

    def body(off_ref, x_ref, w1, b1, w2, b2, w3, b3, w4, b4, out_ref):
        i = pl.program_id(0)
        j0 = i * BLK
        out_ref[...] = jnp.zeros_like(out_ref)
        x = x_ref[...].astype(jnp.bfloat16)
        rows = j0 + lax.broadcasted_iota(jnp.int32, (BLK, 1), 0)

        for e in range(N_EXPERTS):
            lo = off_ref[e]
            hi = off_ref[e + 1]

            @pl.when((j0 < hi) & (j0 + BLK > lo))
            def _compute(e=e, lo=lo, hi=hi):
                h = lax.dot_general(x, w1[e], (((1,), (0,)), ((), ())),
                                    preferred_element_type=jnp.float32)
                h = _celu(h + b1[e]).astype(jnp.bfloat16)
                h = lax.dot_general(h, w2[e], (((1,), (0,)), ((), ())),
                                    preferred_element_type=jnp.float32)
                h = _celu(h + b2[e]).astype(jnp.bfloat16)
                h = lax.dot_general(h, w3[e], (((1,), (0,)), ((), ())),
                                    preferred_element_type=jnp.float32)
                h = _celu(h + b3[e]).astype(jnp.bfloat16)
                y = lax.dot_general(h, w4[e], (((1,), (0,)), ((), ())),
                                    preferred_element_type=jnp.float32)
                y = y + b4[e]
                mask = (rows >= lo) & (rows < hi)
                out_ref[...] += jnp.where(mask, y, 0.0)

    out = pl.pallas_call(
        body,
        grid=(nb,),
        in_specs=[
            pl.BlockSpec(memory_space=pltpu.SMEM),
            pl.BlockSpec((BLK, D), lambda i: (i, 0)),
            pl.BlockSpec(wts[0].shape, lambda i: (0, 0, 0)),
            pl.BlockSpec(bss[0].shape, lambda i: (0, 0, 0)),
            pl.BlockSpec(wts[1].shape, lambda i: (0, 0, 0)),
            pl.BlockSpec(bss[1].shape, lambda i: (0, 0, 0)),
            pl.BlockSpec(wts[2].shape, lambda i: (0, 0, 0)),
            pl.BlockSpec(bss[2].shape, lambda i: (0, 0, 0)),
            pl.BlockSpec(wts[3].shape, lambda i: (0, 0, 0)),
            pl.BlockSpec(bss[3].shape, lambda i: (0, 0, 0)),
        ],
        out_specs=pl.BlockSpec((BLK, 1), lambda i: (i, 0)),
        out_shape=jax.ShapeDtypeStruct((N, 1), jnp.float32),
        compiler_params=pltpu.CompilerParams(
            dimension_semantics=("arbitrary",)),
    )(offsets8, sorted_aev,
      wts[0], bss[0], wts[1], bss[1], wts[2], bss[2], wts[3], bss[3])
    return out


def kernel(species, aev, params):
    n_mol, A = species.shape
    D = aev.shape[-1]
    N = n_mol * A

    sf = species.reshape(-1).astype(jnp.int32)
    aev_flat = aev.reshape(N, D)

    one_hot = (sf[:, None] == jnp.arange(N_EXPERTS, dtype=jnp.int32)[None, :]
               ).astype(jnp.int32)
    csum = jnp.cumsum(one_hot, axis=0)
    counts = csum[-1]
    offsets = jnp.concatenate(
        [jnp.zeros((1,), jnp.int32), jnp.cumsum(counts)]).astype(jnp.int32)
    rank = jnp.sum(one_hot * csum, axis=1) - 1
    dest = offsets[sf] + rank
    dest3 = dest.reshape(NW, (N // NW) // G, G)

    sorted_aev = _sc_scatter_sort(aev_flat, dest3)

    wts = [jnp.stack([p[l][0].T for p in params]).astype(jnp.bfloat16)
           for l in range(4)]
    bss = [jnp.stack([p[l][1] for p in params])[:, None, :] for l in range(4)]
    offsets8 = jnp.concatenate([offsets, jnp.zeros((3,), jnp.int32)])

    y = _tc_moe(sorted_aev, offsets8, wts, bss)
    mol_grp = (N // NW) // (16 * A)
    dest3t = (dest.reshape(NW, mol_grp, 16, A)
              .transpose(0, 1, 3, 2)
              .reshape(NW, (N // NW) // G, G))
    sums = _sc_segment_sum(y.reshape(N), dest3t, n_mol, A)
    return (species, sums)


# speedup vs baseline: 2.1606x; 1.0053x over previous
import functools

import jax
import jax.numpy as jnp
from jax import lax
from jax.experimental import pallas as pl
from jax.experimental.pallas import tpu as pltpu
from jax.experimental.pallas import tpu_sc as plsc

N_EXPERTS = 4
BLK = 512
NW = 32
G = 128


def _sc_scatter_sort(aev_flat, dest3, n_pad):
    N, D = aev_flat.shape
    per_w = N // NW
    n_g = per_w // G
    mesh = plsc.VectorSubcoreMesh(core_axis_name="c", subcore_axis_name="s")

    @functools.partial(
        pl.kernel,
        out_type=jax.ShapeDtypeStruct((n_pad, D), jnp.float32),
        mesh=mesh,
        scratch_types=[
            pltpu.VMEM((n_g, G), jnp.int32),
            pltpu.VMEM((G, D), jnp.float32),
            pltpu.VMEM((G, D), jnp.float32),
            pltpu.SemaphoreType.DMA,
            pltpu.SemaphoreType.DMA,
        ],
    )
    def scatter_kernel(aev_hbm, dest_hbm, out_hbm, idx_v, buf0, buf1,
                       sem_in, sem_out):
        wid = lax.axis_index("s") * 2 + lax.axis_index("c")
        base = wid * per_w
        pltpu.sync_copy(dest_hbm.at[wid], idx_v)
        bufs = (buf0, buf1)
        ins = [pltpu.async_copy(aev_hbm.at[pl.ds(base, G)], buf0, sem_in),
               pltpu.async_copy(aev_hbm.at[pl.ds(base + G, G)], buf1, sem_in)]
        outs = []
        for g in range(n_g):
            ins[g].wait()
            outs.append(pltpu.async_copy(
                bufs[g % 2], out_hbm.at[idx_v.at[g]], sem_out))
            if g + 2 < n_g:
                outs[g].wait()
                ins.append(pltpu.async_copy(
                    aev_hbm.at[pl.ds(base + (g + 2) * G, G)],
                    bufs[g % 2], sem_in))
        outs[n_g - 2].wait()
        outs[n_g - 1].wait()

    return scatter_kernel(aev_flat, dest3)


def _sc_segment_sum(y_flat, dest3t, n_mol, atoms_per_mol, n_atoms):
    N = n_atoms
    per_w = N // NW
    n_g = per_w // G
    mol_per_w = per_w // atoms_per_mol
    n_grp = mol_per_w // 16
    mesh = plsc.VectorSubcoreMesh(core_axis_name="c", subcore_axis_name="s")

    @functools.partial(
        pl.kernel,
        out_type=jax.ShapeDtypeStruct((n_mol,), jnp.float32),
        mesh=mesh,
        scratch_types=[
            pltpu.VMEM((n_g, G), jnp.int32),
            pltpu.VMEM((n_g, G), jnp.float32),
            pltpu.VMEM((mol_per_w,), jnp.float32),
            pltpu.SemaphoreType.DMA,
        ],
    )
    def segsum_kernel(y_hbm, dest_hbm, out_hbm, idx_v, ybuf, out_v, sem):
        wid = lax.axis_index("s") * 2 + lax.axis_index("c")
        pltpu.sync_copy(dest_hbm.at[wid], idx_v)
        cps = [pltpu.async_copy(y_hbm.at[idx_v.at[g]], ybuf.at[g], sem)
               for g in range(n_g)]
        for cp in cps:
            cp.wait()
        for m in range(n_grp):
            acc = jnp.zeros((16,), jnp.float32)
            for t in range(atoms_per_mol):
                off = m * 16 * atoms_per_mol + t * 16
                acc = acc + ybuf[off // G, pl.ds(off % G, 16)]
            out_v[pl.ds(m * 16, 16)] = acc
        pltpu.sync_copy(out_v, out_hbm.at[pl.ds(wid * mol_per_w, mol_per_w)])

    return segsum_kernel(y_flat, dest3t)


def _celu(x):
    p = jnp.exp2(x * 14.426950408889634)
    return jnp.where(x > 0, x, p * 0.1 - 0.1)


def _tc_moe(sorted_aev, offsets8, wts, bss):
    N, D = sorted_aev.shape
    nb = N // BLK

    def body(off_ref, x_ref, w1, b1, w2, b2, w3, b3, w4, b4, out_ref):
        j0 = pl.program_id(0) * BLK
        e = ((j0 >= off_ref[1]).astype(jnp.int32)
             + (j0 >= off_ref[2]).astype(jnp.int32)
             + (j0 >= off_ref[3]).astype(jnp.int32))
        x = x_ref[...].astype(jnp.bfloat16)
        h = lax.dot_general(x, w1[e], (((1,), (0,)), ((), ())),
                            preferred_element_type=jnp.float32)
        h = _celu(h + b1[e]).astype(jnp.bfloat16)
        h = lax.dot_general(h, w2[e], (((1,), (0,)), ((), ())),
                            preferred_element_type=jnp.float32)
        h = _celu(h + b2[e]).astype(jnp.bfloat16)
        h = lax.dot_general(h, w3[e], (((1,), (0,)), ((), ())),
                            preferred_element_type=jnp.float32)
        h = _celu(h + b3[e]).astype(jnp.bfloat16)
        y = lax.dot_general(h, w4[e], (((1,), (0,)), ((), ())),
                            preferred_element_type=jnp.float32)
        out_ref[...] = y + b4[e]

    out = pl.pallas_call(
        body,
        grid=(nb,),
        in_specs=[
            pl.BlockSpec(memory_space=pltpu.SMEM),
            pl.BlockSpec((BLK, D), lambda i: (i, 0)),
            pl.BlockSpec(wts[0].shape, lambda i: (0, 0, 0)),
            pl.BlockSpec(bss[0].shape, lambda i: (0, 0, 0)),
            pl.BlockSpec(wts[1].shape, lambda i: (0, 0, 0)),
            pl.BlockSpec(bss[1].shape, lambda i: (0, 0, 0)),
            pl.BlockSpec(wts[2].shape, lambda i: (0, 0, 0)),
            pl.BlockSpec(bss[2].shape, lambda i: (0, 0, 0)),
            pl.BlockSpec(wts[3].shape, lambda i: (0, 0, 0)),
            pl.BlockSpec(bss[3].shape, lambda i: (0, 0, 0)),
        ],
        out_specs=pl.BlockSpec((BLK, 1), lambda i: (i, 0)),
        out_shape=jax.ShapeDtypeStruct((N, 1), jnp.float32),
        compiler_params=pltpu.CompilerParams(
            dimension_semantics=("arbitrary",)),
    )(offsets8, sorted_aev,
      wts[0], bss[0], wts[1], bss[1], wts[2], bss[2], wts[3], bss[3])
    return out


def kernel(species, aev, params):
    n_mol, A = species.shape
    D = aev.shape[-1]
    N = n_mol * A

    sf = species.reshape(-1).astype(jnp.int32)
    aev_flat = aev.reshape(N, D)

    one_hot = (sf[:, None] == jnp.arange(N_EXPERTS, dtype=jnp.int32)[None, :]
               ).astype(jnp.int32)
    csum = jnp.cumsum(one_hot, axis=0)
    counts = csum[-1]
    counts_pad = ((counts + BLK - 1) // BLK) * BLK
    offsets = jnp.concatenate(
        [jnp.zeros((1,), jnp.int32), jnp.cumsum(counts_pad)]).astype(jnp.int32)
    rank = jnp.sum(one_hot * csum, axis=1) - 1
    dest = offsets[sf] + rank
    dest3 = dest.reshape(NW, (N // NW) // G, G)
    n_pad = N + N_EXPERTS * BLK

    sorted_aev = _sc_scatter_sort(aev_flat, dest3, n_pad)

    wts = [jnp.stack([p[l][0].T for p in params]).astype(jnp.bfloat16)
           for l in range(4)]
    bss = [jnp.stack([p[l][1] for p in params])[:, None, :] for l in range(4)]
    offsets8 = jnp.concatenate([offsets, jnp.zeros((3,), jnp.int32)])

    y = _tc_moe(sorted_aev, offsets8, wts, bss)
    mol_grp = (N // NW) // (16 * A)
    dest3t = (dest.reshape(NW, mol_grp, 16, A)
              .transpose(0, 1, 3, 2)
              .reshape(NW, (N // NW) // G, G))
    sums = _sc_segment_sum(y.reshape(n_pad), dest3t, n_mol, A, N)
    return (species, sums)


# speedup vs baseline: 2.5759x; 1.1922x over previous
import functools

import jax
import jax.numpy as jnp
from jax import lax
from jax.experimental import pallas as pl
from jax.experimental.pallas import tpu as pltpu
from jax.experimental.pallas import tpu_sc as plsc

N_EXPERTS = 4
BLK = 1024
NW = 32
G = 128


def _sc_scatter_sort(aev_flat, dest3, n_pad):
    N, D = aev_flat.shape
    per_w = N // NW
    n_g = per_w // G
    mesh = plsc.VectorSubcoreMesh(core_axis_name="c", subcore_axis_name="s")

    @functools.partial(
        pl.kernel,
        out_type=jax.ShapeDtypeStruct((n_pad, D), jnp.float32),
        mesh=mesh,
        scratch_types=[
            pltpu.VMEM((n_g, G), jnp.int32),
            pltpu.VMEM((G, D), jnp.float32),
            pltpu.VMEM((G, D), jnp.float32),
            pltpu.SemaphoreType.DMA,
            pltpu.SemaphoreType.DMA,
        ],
    )
    def scatter_kernel(aev_hbm, dest_hbm, out_hbm, idx_v, buf0, buf1,
                       sem_in, sem_out):
        wid = lax.axis_index("s") * 2 + lax.axis_index("c")
        base = wid * per_w
        pltpu.sync_copy(dest_hbm.at[wid], idx_v)
        bufs = (buf0, buf1)
        ins = [pltpu.async_copy(aev_hbm.at[pl.ds(base, G)], buf0, sem_in),
               pltpu.async_copy(aev_hbm.at[pl.ds(base + G, G)], buf1, sem_in)]
        outs = []
        for g in range(n_g):
            ins[g].wait()
            outs.append(pltpu.async_copy(
                bufs[g % 2], out_hbm.at[idx_v.at[g]], sem_out))
            if g + 2 < n_g:
                outs[g].wait()
                ins.append(pltpu.async_copy(
                    aev_hbm.at[pl.ds(base + (g + 2) * G, G)],
                    bufs[g % 2], sem_in))
        outs[n_g - 2].wait()
        outs[n_g - 1].wait()

    return scatter_kernel(aev_flat, dest3)


def _sc_segment_sum(y_flat, dest3t, n_mol, atoms_per_mol, n_atoms):
    N = n_atoms
    per_w = N // NW
    n_g = per_w // G
    mol_per_w = per_w // atoms_per_mol
    n_grp = mol_per_w // 16
    mesh = plsc.VectorSubcoreMesh(core_axis_name="c", subcore_axis_name="s")

    @functools.partial(
        pl.kernel,
        out_type=jax.ShapeDtypeStruct((n_mol,), jnp.float32),
        mesh=mesh,
        scratch_types=[
            pltpu.VMEM((n_g, G), jnp.int32),
            pltpu.VMEM((n_g, G), jnp.float32),
            pltpu.VMEM((mol_per_w,), jnp.float32),
            pltpu.SemaphoreType.DMA,
        ],
    )
    def segsum_kernel(y_hbm, dest_hbm, out_hbm, idx_v, ybuf, out_v, sem):
        wid = lax.axis_index("s") * 2 + lax.axis_index("c")
        pltpu.sync_copy(dest_hbm.at[wid], idx_v)
        cps = [pltpu.async_copy(y_hbm.at[idx_v.at[g]], ybuf.at[g], sem)
               for g in range(n_g)]
        for cp in cps:
            cp.wait()
        for m in range(n_grp):
            acc = jnp.zeros((16,), jnp.float32)
            for t in range(atoms_per_mol):
                off = m * 16 * atoms_per_mol + t * 16
                acc = acc + ybuf[off // G, pl.ds(off % G, 16)]
            out_v[pl.ds(m * 16, 16)] = acc
        pltpu.sync_copy(out_v, out_hbm.at[pl.ds(wid * mol_per_w, mol_per_w)])

    return segsum_kernel(y_flat, dest3t)


def _celu(x):
    p = jnp.exp2(x * 14.426950408889634)
    return jnp.where(x > 0, x, p * 0.1 - 0.1)


def _tc_moe(sorted_aev, offsets8, wts, bss):
    N, D = sorted_aev.shape
    nb = N // BLK

    def body(off_ref, x_ref, w1, b1, w2, b2, w3, b3, w4, b4, out_ref):
        j0 = pl.program_id(0) * BLK
        e = ((j0 >= off_ref[1]).astype(jnp.int32)
             + (j0 >= off_ref[2]).astype(jnp.int32)
             + (j0 >= off_ref[3]).astype(jnp.int32))
        x = x_ref[...].astype(jnp.bfloat16)
        h = lax.dot_general(x, w1[e], (((1,), (0,)), ((), ())),
                            preferred_element_type=jnp.float32)
        h = _celu(h + b1[e]).astype(jnp.bfloat16)
        h = lax.dot_general(h, w2[e], (((1,), (0,)), ((), ())),
                            preferred_element_type=jnp.float32)
        h = _celu(h + b2[e]).astype(jnp.bfloat16)
        h = lax.dot_general(h, w3[e], (((1,), (0,)), ((), ())),
                            preferred_element_type=jnp.float32)
        h = _celu(h + b3[e]).astype(jnp.bfloat16)
        y = lax.dot_general(h, w4[e], (((1,), (0,)), ((), ())),
                            preferred_element_type=jnp.float32)
        out_ref[...] = y + b4[e]

    out = pl.pallas_call(
        body,
        grid=(nb,),
        in_specs=[
            pl.BlockSpec(memory_space=pltpu.SMEM),
            pl.BlockSpec((BLK, D), lambda i: (i, 0)),
            pl.BlockSpec(wts[0].shape, lambda i: (0, 0, 0)),
            pl.BlockSpec(bss[0].shape, lambda i: (0, 0, 0)),
            pl.BlockSpec(wts[1].shape, lambda i: (0, 0, 0)),
            pl.BlockSpec(bss[1].shape, lambda i: (0, 0, 0)),
            pl.BlockSpec(wts[2].shape, lambda i: (0, 0, 0)),
            pl.BlockSpec(bss[2].shape, lambda i: (0, 0, 0)),
            pl.BlockSpec(wts[3].shape, lambda i: (0, 0, 0)),
            pl.BlockSpec(bss[3].shape, lambda i: (0, 0, 0)),
        ],
        out_specs=pl.BlockSpec((BLK, 1), lambda i: (i, 0)),
        out_shape=jax.ShapeDtypeStruct((N, 1), jnp.float32),
        compiler_params=pltpu.CompilerParams(
            dimension_semantics=("arbitrary",)),
    )(offsets8, sorted_aev,
      wts[0], bss[0], wts[1], bss[1], wts[2], bss[2], wts[3], bss[3])
    return out


def kernel(species, aev, params):
    n_mol, A = species.shape
    D = aev.shape[-1]
    N = n_mol * A

    sf = species.reshape(-1).astype(jnp.int32)
    aev_flat = aev.reshape(N, D)

    one_hot = (sf[:, None] == jnp.arange(N_EXPERTS, dtype=jnp.int32)[None, :]
               ).astype(jnp.int32)
    csum = jnp.cumsum(one_hot, axis=0)
    counts = csum[-1]
    counts_pad = ((counts + BLK - 1) // BLK) * BLK
    offsets = jnp.concatenate(
        [jnp.zeros((1,), jnp.int32), jnp.cumsum(counts_pad)]).astype(jnp.int32)
    rank = jnp.sum(one_hot * csum, axis=1) - 1
    dest = offsets[sf] + rank
    dest3 = dest.reshape(NW, (N // NW) // G, G)
    n_pad = N + N_EXPERTS * BLK

    sorted_aev = _sc_scatter_sort(aev_flat, dest3, n_pad)

    wts = [jnp.stack([p[l][0].T for p in params]).astype(jnp.bfloat16)
           for l in range(4)]
    bss = [jnp.stack([p[l][1] for p in params])[:, None, :] for l in range(4)]
    offsets8 = jnp.concatenate([offsets, jnp.zeros((3,), jnp.int32)])

    y = _tc_moe(sorted_aev, offsets8, wts, bss)
    mol_grp = (N // NW) // (16 * A)
    dest3t = (dest.reshape(NW, mol_grp, 16, A)
              .transpose(0, 1, 3, 2)
              .reshape(NW, (N // NW) // G, G))
    sums = _sc_segment_sum(y.reshape(n_pad), dest3t, n_mol, A, N)
    return (species, sums)


# speedup vs baseline: 2.7124x; 1.0530x over previous
import functools

import jax
import jax.numpy as jnp
from jax import lax
from jax.experimental import pallas as pl
from jax.experimental.pallas import tpu as pltpu
from jax.experimental.pallas import tpu_sc as plsc

N_EXPERTS = 4
BLK = 2048
NW = 32
G = 128


def _sc_scatter_sort(aev_flat, dest3, n_pad):
    N, D = aev_flat.shape
    per_w = N // NW
    n_g = per_w // G
    mesh = plsc.VectorSubcoreMesh(core_axis_name="c", subcore_axis_name="s")

    @functools.partial(
        pl.kernel,
        out_type=jax.ShapeDtypeStruct((n_pad, D), jnp.float32),
        mesh=mesh,
        scratch_types=[
            pltpu.VMEM((n_g, G), jnp.int32),
            pltpu.VMEM((G, D), jnp.float32),
            pltpu.VMEM((G, D), jnp.float32),
            pltpu.SemaphoreType.DMA,
            pltpu.SemaphoreType.DMA,
        ],
    )
    def scatter_kernel(aev_hbm, dest_hbm, out_hbm, idx_v, buf0, buf1,
                       sem_in, sem_out):
        wid = lax.axis_index("s") * 2 + lax.axis_index("c")
        base = wid * per_w
        pltpu.sync_copy(dest_hbm.at[wid], idx_v)
        bufs = (buf0, buf1)
        ins = [pltpu.async_copy(aev_hbm.at[pl.ds(base, G)], buf0, sem_in),
               pltpu.async_copy(aev_hbm.at[pl.ds(base + G, G)], buf1, sem_in)]
        outs = []
        for g in range(n_g):
            ins[g].wait()
            outs.append(pltpu.async_copy(
                bufs[g % 2], out_hbm.at[idx_v.at[g]], sem_out))
            if g + 2 < n_g:
                outs[g].wait()
                ins.append(pltpu.async_copy(
                    aev_hbm.at[pl.ds(base + (g + 2) * G, G)],
                    bufs[g % 2], sem_in))
        outs[n_g - 2].wait()
        outs[n_g - 1].wait()

    return scatter_kernel(aev_flat, dest3)


def _sc_segment_sum(y_flat, dest3t, n_mol, atoms_per_mol, n_atoms):
    N = n_atoms
    per_w = N // NW
    n_g = per_w // G
    mol_per_w = per_w // atoms_per_mol
    n_grp = mol_per_w // 16
    mesh = plsc.VectorSubcoreMesh(core_axis_name="c", subcore_axis_name="s")

    @functools.partial(
        pl.kernel,
        out_type=jax.ShapeDtypeStruct((n_mol,), jnp.float32),
        mesh=mesh,
        scratch_types=[
            pltpu.VMEM((n_g, G), jnp.int32),
            pltpu.VMEM((n_g, G), jnp.float32),
            pltpu.VMEM((mol_per_w,), jnp.float32),
            pltpu.SemaphoreType.DMA,
        ],
    )
    def segsum_kernel(y_hbm, dest_hbm, out_hbm, idx_v, ybuf, out_v, sem):
        wid = lax.axis_index("s") * 2 + lax.axis_index("c")
        pltpu.sync_copy(dest_hbm.at[wid], idx_v)
        cps = [pltpu.async_copy(y_hbm.at[idx_v.at[g]], ybuf.at[g], sem)
               for g in range(n_g)]
        for cp in cps:
            cp.wait()
        for m in range(n_grp):
            acc = jnp.zeros((16,), jnp.float32)
            for t in range(atoms_per_mol):
                off = m * 16 * atoms_per_mol + t * 16
                acc = acc + ybuf[off // G, pl.ds(off % G, 16)]
            out_v[pl.ds(m * 16, 16)] = acc
        pltpu.sync_copy(out_v, out_hbm.at[pl.ds(wid * mol_per_w, mol_per_w)])

    return segsum_kernel(y_flat, dest3t)


def _celu(x):
    p = jnp.exp2(x * 14.426950408889634)
    return jnp.where(x > 0, x, p * 0.1 - 0.1)


def _tc_moe(sorted_aev, offsets8, wts, bss):
    N, D = sorted_aev.shape
    nb = N // BLK

    def body(off_ref, x_ref, w1, b1, w2, b2, w3, b3, w4, b4, out_ref):
        j0 = pl.program_id(0) * BLK
        e = ((j0 >= off_ref[1]).astype(jnp.int32)
             + (j0 >= off_ref[2]).astype(jnp.int32)
             + (j0 >= off_ref[3]).astype(jnp.int32))
        x = x_ref[...].astype(jnp.bfloat16)
        h = lax.dot_general(x, w1[e], (((1,), (0,)), ((), ())),
                            preferred_element_type=jnp.float32)
        h = _celu(h + b1[e]).astype(jnp.bfloat16)
        h = lax.dot_general(h, w2[e], (((1,), (0,)), ((), ())),
                            preferred_element_type=jnp.float32)
        h = _celu(h + b2[e]).astype(jnp.bfloat16)
        h = lax.dot_general(h, w3[e], (((1,), (0,)), ((), ())),
                            preferred_element_type=jnp.float32)
        h = _celu(h + b3[e]).astype(jnp.bfloat16)
        y = lax.dot_general(h, w4[e], (((1,), (0,)), ((), ())),
                            preferred_element_type=jnp.float32)
        out_ref[...] = y + b4[e]

    out = pl.pallas_call(
        body,
        grid=(nb,),
        in_specs=[
            pl.BlockSpec(memory_space=pltpu.SMEM),
            pl.BlockSpec((BLK, D), lambda i: (i, 0)),
            pl.BlockSpec(wts[0].shape, lambda i: (0, 0, 0)),
            pl.BlockSpec(bss[0].shape, lambda i: (0, 0, 0)),
            pl.BlockSpec(wts[1].shape, lambda i: (0, 0, 0)),
            pl.BlockSpec(bss[1].shape, lambda i: (0, 0, 0)),
            pl.BlockSpec(wts[2].shape, lambda i: (0, 0, 0)),
            pl.BlockSpec(bss[2].shape, lambda i: (0, 0, 0)),
            pl.BlockSpec(wts[3].shape, lambda i: (0, 0, 0)),
            pl.BlockSpec(bss[3].shape, lambda i: (0, 0, 0)),
        ],
        out_specs=pl.BlockSpec((BLK, 1), lambda i: (i, 0)),
        out_shape=jax.ShapeDtypeStruct((N, 1), jnp.float32),
        compiler_params=pltpu.CompilerParams(
            dimension_semantics=("arbitrary",)),
    )(offsets8, sorted_aev,
      wts[0], bss[0], wts[1], bss[1], wts[2], bss[2], wts[3], bss[3])
    return out


def kernel(species, aev, params):
    n_mol, A = species.shape
    D = aev.shape[-1]
    N = n_mol * A

    sf = species.reshape(-1).astype(jnp.int32)
    aev_flat = aev.reshape(N, D)

    one_hot = (sf[:, None] == jnp.arange(N_EXPERTS, dtype=jnp.int32)[None, :]
               ).astype(jnp.int32)
    csum = jnp.cumsum(one_hot, axis=0)
    counts = csum[-1]
    counts_pad = ((counts + BLK - 1) // BLK) * BLK
    offsets = jnp.concatenate(
        [jnp.zeros((1,), jnp.int32), jnp.cumsum(counts_pad)]).astype(jnp.int32)
    rank = jnp.sum(one_hot * csum, axis=1) - 1
    dest = offsets[sf] + rank
    dest3 = dest.reshape(NW, (N // NW) // G, G)
    n_pad = N + N_EXPERTS * BLK

    sorted_aev = _sc_scatter_sort(aev_flat, dest3, n_pad)

    wts = [jnp.stack([p[l][0].T for p in params]).astype(jnp.bfloat16)
           for l in range(4)]
    bss = [jnp.stack([p[l][1] for p in params])[:, None, :] for l in range(4)]
    offsets8 = jnp.concatenate([offsets, jnp.zeros((3,), jnp.int32)])

    y = _tc_moe(sorted_aev, offsets8, wts, bss)
    mol_grp = (N // NW) // (16 * A)
    dest3t = (dest.reshape(NW, mol_grp, 16, A)
              .transpose(0, 1, 3, 2)
              .reshape(NW, (N // NW) // G, G))
    sums = _sc_segment_sum(y.reshape(n_pad), dest3t, n_mol, A, N)
    return (species, sums)
